# Initial kernel scaffold; baseline (speedup 1.0000x reference)
import jax
import jax.numpy as jnp
from jax.experimental import pallas as pl


def kernel(surrounds, centers, targets, emb_surround, emb_center, W_h, b_h, W_o, b_o):
    raise NotImplementedError("write your pallas kernel here")


# speedup vs baseline: 10.4289x; 10.4289x over previous
import functools

import jax
import jax.numpy as jnp
from jax import lax
from jax.experimental import pallas as pl
from jax.experimental.pallas import tpu as pltpu
from jax.experimental.pallas import tpu_sc as plsc

V = 50003
D = 300
DP = 304
DH = 256
NLAB = 17
NLABP = 128
B = 16384
L = 50
LP1 = 51

NC = 2
NS = 16
NW = NC * NS
BPW = B // NW
SPC = 2
CW = SPC * LP1
NCHUNK = BPW // SPC
NVR = DP // 16


def _sc_pool_body(tbl_hbm, idx_hbm, out_hbm, idx_v, rows_v, stage_v, gsem):
    wid = lax.axis_index("s") * NC + lax.axis_index("c")

    pltpu.sync_copy(idx_hbm.at[wid], idx_v)

    def chunk_body(g, carry):
        pltpu.async_copy(tbl_hbm.at[idx_v.at[g]], rows_v, gsem).wait()
        for sloc in range(SPC):
            base_r = sloc * LP1
            acc0 = tuple(rows_v[base_r, pl.ds(16 * k, 16)] for k in range(NVR))

            def row_body(r, acc):
                return tuple(
                    acc[k] + rows_v[base_r + r, pl.ds(16 * k, 16)]
                    for k in range(NVR)
                )

            acc = lax.fori_loop(1, LP1, row_body, acc0)
            for k in range(NVR):
                stage_v[sloc, pl.ds(16 * k, 16)] = acc[k]
        base_s = wid * BPW + g * SPC
        pltpu.sync_copy(stage_v, out_hbm.at[pl.ds(base_s, SPC)])
        return carry

    lax.fori_loop(0, NCHUNK, chunk_body, 0)


def _make_sc_pool():
    mesh = plsc.VectorSubcoreMesh(core_axis_name="c", subcore_axis_name="s")
    return pl.kernel(
        _sc_pool_body,
        out_type=jax.ShapeDtypeStruct((B, DP), jnp.float32),
        mesh=mesh,
        scratch_types=[
            pltpu.VMEM((NCHUNK, CW), jnp.int32),
            pltpu.VMEM((CW, DP), jnp.float32),
            pltpu.VMEM((SPC, DP), jnp.float32),
            pltpu.SemaphoreType.DMA,
        ],
        compiler_params=pltpu.CompilerParams(use_tc_tiling_on_sc=False),
    )


BLK = 1024
NBLK = B // BLK


def _head_body(x_ref, wh_ref, bh_ref, wo_ref, bo_ref, tgt_ref,
               loss_ref, cnt_ref):
    i = pl.program_id(0)
    x = x_ref[...] * jnp.float32(1.0 / LP1)
    h = jnp.tanh(
        jnp.dot(x, wh_ref[...], preferred_element_type=jnp.float32)
        + bh_ref[...]
    )
    logits = (
        jnp.dot(h, wo_ref[...], preferred_element_type=jnp.float32)
        + bo_ref[...]
    )
    col = lax.broadcasted_iota(jnp.int32, (BLK, NLABP), 1)
    lm = jnp.where(col < NLAB, logits, jnp.float32(-1e30))
    mx = jnp.max(lm, axis=1, keepdims=True)
    lse = jnp.log(jnp.sum(jnp.exp(lm - mx), axis=1, keepdims=True)) + mx
    tgt = tgt_ref[0, 0, :]
    tlogit = jnp.sum(
        jnp.where(col == tgt[:, None], lm, jnp.float32(0.0)),
        axis=1, keepdims=True,
    )
    loss_part = jnp.sum(lse - tlogit)
    pred = jnp.min(jnp.where(lm >= mx, col, NLABP), axis=1)
    cnt_part = jnp.sum((pred == tgt).astype(jnp.float32))

    @pl.when(i == 0)
    def _():
        loss_ref[...] = jnp.zeros_like(loss_ref)
        cnt_ref[...] = jnp.zeros_like(cnt_ref)

    loss_ref[...] += loss_part
    cnt_ref[...] += cnt_part


def _make_head(interpret=False):
    return pl.pallas_call(
        _head_body,
        grid=(NBLK,),
        in_specs=[
            pl.BlockSpec((BLK, DP), lambda i: (i, 0)),
            pl.BlockSpec((DP, DH), lambda i: (0, 0)),
            pl.BlockSpec((1, DH), lambda i: (0, 0)),
            pl.BlockSpec((DH, NLABP), lambda i: (0, 0)),
            pl.BlockSpec((1, NLABP), lambda i: (0, 0)),
            pl.BlockSpec((1, 1, BLK), lambda i: (i, 0, 0)),
        ],
        out_specs=[
            pl.BlockSpec((1, 1), lambda i: (0, 0)),
            pl.BlockSpec((1, 1), lambda i: (0, 0)),
        ],
        out_shape=[
            jax.ShapeDtypeStruct((1, 1), jnp.float32),
            jax.ShapeDtypeStruct((1, 1), jnp.float32),
        ],
        interpret=interpret,
    )


@jax.jit
def kernel(surrounds, centers, targets, emb_surround, emb_center,
           W_h, b_h, W_o, b_o):
    tbl = jnp.concatenate(
        [
            jnp.pad(emb_surround, ((0, 0), (0, DP - D))),
            jnp.pad(emb_center, ((0, 0), (0, DP - D))),
        ],
        axis=0,
    )
    idx = jnp.concatenate(
        [surrounds.astype(jnp.int32), centers.astype(jnp.int32)[:, None] + V],
        axis=1,
    ).reshape(NW, NCHUNK, CW)

    pooled = _make_sc_pool()(tbl, idx)

    wh = jnp.pad(W_h, ((0, DP - D), (0, 0)))
    wo = jnp.pad(W_o, ((0, 0), (0, NLABP - NLAB)))
    bo = jnp.pad(b_o, (0, NLABP - NLAB))
    tgt = targets.astype(jnp.int32).reshape(NBLK, 1, BLK)

    loss, cnt = _make_head()(
        pooled, wh, b_h[None, :], wo, bo[None, :], tgt
    )
    return loss[0, 0], cnt[0, 0] / jnp.float32(B)


# speedup vs baseline: 13.4896x; 1.2935x over previous
import functools

import jax
import jax.numpy as jnp
from jax import lax
from jax.experimental import pallas as pl
from jax.experimental.pallas import tpu as pltpu
from jax.experimental.pallas import tpu_sc as plsc

V = 50003
D = 300
DP = 304
DH = 256
NLAB = 17
NLABP = 128
B = 16384
L = 50
LP1 = 51

NC = 2
NS = 16
NW = NC * NS
BPW = B // NW
SPC = 2
CW = SPC * LP1
NCHUNK = BPW // SPC
NVR = DP // 16


def _sc_pool_body(tbl_hbm, idx_hbm, out_hbm, idx_v,
                  rows0, rows1, stage0, stage1, gsem0, gsem1):
    wid = lax.axis_index("s") * NC + lax.axis_index("c")
    rows = (rows0, rows1)
    stage = (stage0, stage1)
    gsem = (gsem0, gsem1)

    pltpu.sync_copy(idx_hbm.at[wid], idx_v)

    def start(c, b):
        pltpu.async_copy(tbl_hbm.at[idx_v.at[c]], rows[b], gsem[b])

    def wait(b):
        pltpu.make_async_copy(tbl_hbm.at[pl.ds(0, CW)], rows[b], gsem[b]).wait()

    def process(c, b):
        for sloc in range(SPC):
            base_r = sloc * LP1
            acc0 = tuple(
                rows[b][base_r, pl.ds(16 * k, 16)] for k in range(NVR)
            )

            def row_body(r, acc):
                return tuple(
                    acc[k] + rows[b][base_r + r, pl.ds(16 * k, 16)]
                    for k in range(NVR)
                )

            acc = lax.fori_loop(1, LP1, row_body, acc0)
            for k in range(NVR):
                stage[b][sloc, pl.ds(16 * k, 16)] = acc[k]
        pltpu.sync_copy(stage[b], out_hbm.at[pl.ds(wid * BPW + c * SPC, SPC)])

    start(0, 0)
    start(1, 1)

    def outer(i, carry):
        c = 2 * i
        wait(0)
        process(c, 0)
        start(c + 2, 0)
        wait(1)
        process(c + 1, 1)
        start(c + 3, 1)
        return carry

    lax.fori_loop(0, NCHUNK // 2 - 1, outer, 0)
    wait(0)
    process(NCHUNK - 2, 0)
    wait(1)
    process(NCHUNK - 1, 1)


def _make_sc_pool():
    mesh = plsc.VectorSubcoreMesh(core_axis_name="c", subcore_axis_name="s")
    return pl.kernel(
        _sc_pool_body,
        out_type=jax.ShapeDtypeStruct((B, DP), jnp.float32),
        mesh=mesh,
        scratch_types=[
            pltpu.VMEM((NCHUNK, CW), jnp.int32),
            pltpu.VMEM((CW, DP), jnp.float32),
            pltpu.VMEM((CW, DP), jnp.float32),
            pltpu.VMEM((SPC, DP), jnp.float32),
            pltpu.VMEM((SPC, DP), jnp.float32),
            pltpu.SemaphoreType.DMA,
            pltpu.SemaphoreType.DMA,
        ],
        compiler_params=pltpu.CompilerParams(use_tc_tiling_on_sc=False),
    )


BLK = 1024
NBLK = B // BLK


def _head_body(x_ref, wh_ref, bh_ref, wo_ref, bo_ref, tgt_ref,
               loss_ref, cnt_ref):
    i = pl.program_id(0)
    x = x_ref[...] * jnp.float32(1.0 / LP1)
    h = jnp.tanh(
        jnp.dot(x, wh_ref[...], preferred_element_type=jnp.float32)
        + bh_ref[...]
    )
    logits = (
        jnp.dot(h, wo_ref[...], preferred_element_type=jnp.float32)
        + bo_ref[...]
    )
    col = lax.broadcasted_iota(jnp.int32, (BLK, NLABP), 1)
    lm = jnp.where(col < NLAB, logits, jnp.float32(-1e30))
    mx = jnp.max(lm, axis=1, keepdims=True)
    lse = jnp.log(jnp.sum(jnp.exp(lm - mx), axis=1, keepdims=True)) + mx
    tgt = tgt_ref[0, 0, :]
    tlogit = jnp.sum(
        jnp.where(col == tgt[:, None], lm, jnp.float32(0.0)),
        axis=1, keepdims=True,
    )
    loss_part = jnp.sum(lse - tlogit)
    pred = jnp.min(jnp.where(lm >= mx, col, NLABP), axis=1)
    cnt_part = jnp.sum((pred == tgt).astype(jnp.float32))

    @pl.when(i == 0)
    def _():
        loss_ref[...] = jnp.zeros_like(loss_ref)
        cnt_ref[...] = jnp.zeros_like(cnt_ref)

    loss_ref[...] += loss_part
    cnt_ref[...] += cnt_part


def _make_head(interpret=False):
    return pl.pallas_call(
        _head_body,
        grid=(NBLK,),
        in_specs=[
            pl.BlockSpec((BLK, DP), lambda i: (i, 0)),
            pl.BlockSpec((DP, DH), lambda i: (0, 0)),
            pl.BlockSpec((1, DH), lambda i: (0, 0)),
            pl.BlockSpec((DH, NLABP), lambda i: (0, 0)),
            pl.BlockSpec((1, NLABP), lambda i: (0, 0)),
            pl.BlockSpec((1, 1, BLK), lambda i: (i, 0, 0)),
        ],
        out_specs=[
            pl.BlockSpec((1, 1), lambda i: (0, 0)),
            pl.BlockSpec((1, 1), lambda i: (0, 0)),
        ],
        out_shape=[
            jax.ShapeDtypeStruct((1, 1), jnp.float32),
            jax.ShapeDtypeStruct((1, 1), jnp.float32),
        ],
        interpret=interpret,
    )


@jax.jit
def kernel(surrounds, centers, targets, emb_surround, emb_center,
           W_h, b_h, W_o, b_o):
    tbl = jnp.concatenate(
        [
            jnp.pad(emb_surround, ((0, 0), (0, DP - D))),
            jnp.pad(emb_center, ((0, 0), (0, DP - D))),
        ],
        axis=0,
    )
    idx = jnp.concatenate(
        [surrounds.astype(jnp.int32), centers.astype(jnp.int32)[:, None] + V],
        axis=1,
    ).reshape(NW, NCHUNK, CW)

    pooled = _make_sc_pool()(tbl, idx)

    wh = jnp.pad(W_h, ((0, DP - D), (0, 0)))
    wo = jnp.pad(W_o, ((0, 0), (0, NLABP - NLAB)))
    bo = jnp.pad(b_o, (0, NLABP - NLAB))
    tgt = targets.astype(jnp.int32).reshape(NBLK, 1, BLK)

    loss, cnt = _make_head()(
        pooled, wh, b_h[None, :], wo, bo[None, :], tgt
    )
    return loss[0, 0], cnt[0, 0] / jnp.float32(B)


# speedup vs baseline: 18.9380x; 1.4039x over previous
import functools

import jax
import jax.numpy as jnp
from jax import lax
from jax.experimental import pallas as pl
from jax.experimental.pallas import tpu as pltpu
from jax.experimental.pallas import tpu_sc as plsc

V = 50003
D = 300
DP = 304
DH = 256
NLAB = 17
NLABP = 128
B = 16384
L = 50
LP1 = 51

NC = 2
NS = 16
NW = NC * NS
BPW = B // NW
SPC = 2
CW = SPC * LP1
NCHUNK = BPW // SPC
NVR = DP // 16


def _sc_pool_body(sur_hbm, cen_hbm, sidx_hbm, cidx_hbm, out_hbm,
                  sidx_v, cidx_v, rows0, rows1, stage0, stage1,
                  gsem0, gsem1):
    wid = lax.axis_index("s") * NC + lax.axis_index("c")
    rows = (rows0, rows1)
    stage = (stage0, stage1)
    gsem = (gsem0, gsem1)

    pltpu.sync_copy(sidx_hbm.at[wid], sidx_v)
    pltpu.sync_copy(cidx_hbm.at[wid], cidx_v)

    def start(c, b):
        pltpu.async_copy(sur_hbm.at[sidx_v.at[c]],
                         rows[b].at[pl.ds(0, SPC * L)], gsem[b])
        pltpu.async_copy(cen_hbm.at[cidx_v.at[c]],
                         rows[b].at[pl.ds(SPC * L, SPC)], gsem[b])

    def wait(b):
        pltpu.make_async_copy(sur_hbm.at[pl.ds(0, SPC * L)],
                              rows[b].at[pl.ds(0, SPC * L)], gsem[b]).wait()
        pltpu.make_async_copy(cen_hbm.at[pl.ds(0, SPC)],
                              rows[b].at[pl.ds(SPC * L, SPC)], gsem[b]).wait()

    def process(c, b):
        for sloc in range(SPC):
            base_r = sloc * L
            acc0 = tuple(
                rows[b][SPC * L + sloc, pl.ds(16 * k, 16)]
                for k in range(NVR)
            )

            def row_body(r, acc):
                return tuple(
                    acc[k] + rows[b][base_r + r, pl.ds(16 * k, 16)]
                    for k in range(NVR)
                )

            acc = lax.fori_loop(0, L, row_body, acc0)
            for k in range(NVR):
                stage[b][sloc, pl.ds(16 * k, 16)] = acc[k]
        pltpu.sync_copy(stage[b], out_hbm.at[pl.ds(wid * BPW + c * SPC, SPC)])

    start(0, 0)
    start(1, 1)

    def outer(i, carry):
        c = 2 * i
        wait(0)
        process(c, 0)
        start(c + 2, 0)
        wait(1)
        process(c + 1, 1)
        start(c + 3, 1)
        return carry

    lax.fori_loop(0, NCHUNK // 2 - 1, outer, 0)
    wait(0)
    process(NCHUNK - 2, 0)
    wait(1)
    process(NCHUNK - 1, 1)


def _make_sc_pool():
    mesh = plsc.VectorSubcoreMesh(core_axis_name="c", subcore_axis_name="s")
    return pl.kernel(
        _sc_pool_body,
        out_type=jax.ShapeDtypeStruct((B, DP), jnp.float32),
        mesh=mesh,
        scratch_types=[
            pltpu.VMEM((NCHUNK, SPC * L), jnp.int32),
            pltpu.VMEM((NCHUNK, SPC), jnp.int32),
            pltpu.VMEM((CW, DP), jnp.float32),
            pltpu.VMEM((CW, DP), jnp.float32),
            pltpu.VMEM((SPC, DP), jnp.float32),
            pltpu.VMEM((SPC, DP), jnp.float32),
            pltpu.SemaphoreType.DMA,
            pltpu.SemaphoreType.DMA,
        ],
        compiler_params=pltpu.CompilerParams(use_tc_tiling_on_sc=False),
    )


PBLK = 4096
NPBLK = -(-V // PBLK)


def _pad_body(a_ref, b_ref, ap_ref, bp_ref):
    z = jnp.zeros((PBLK, DP - D), jnp.float32)
    ap_ref[...] = jnp.concatenate([a_ref[...], z], axis=1)
    bp_ref[...] = jnp.concatenate([b_ref[...], z], axis=1)


def _make_pad():
    return pl.pallas_call(
        _pad_body,
        grid=(NPBLK,),
        in_specs=[
            pl.BlockSpec((PBLK, D), lambda i: (i, 0)),
            pl.BlockSpec((PBLK, D), lambda i: (i, 0)),
        ],
        out_specs=[
            pl.BlockSpec((PBLK, DP), lambda i: (i, 0)),
            pl.BlockSpec((PBLK, DP), lambda i: (i, 0)),
        ],
        out_shape=[
            jax.ShapeDtypeStruct((V, DP), jnp.float32),
            jax.ShapeDtypeStruct((V, DP), jnp.float32),
        ],
    )


BLK = 1024
NBLK = B // BLK


def _head_body(x_ref, wh_ref, bh_ref, wo_ref, bo_ref, tgt_ref,
               loss_ref, cnt_ref):
    i = pl.program_id(0)
    x = x_ref[...] * jnp.float32(1.0 / LP1)
    h = jnp.tanh(
        jnp.dot(x, wh_ref[...], preferred_element_type=jnp.float32)
        + bh_ref[...]
    )
    logits = (
        jnp.dot(h, wo_ref[...], preferred_element_type=jnp.float32)
        + bo_ref[...]
    )
    col = lax.broadcasted_iota(jnp.int32, (BLK, NLABP), 1)
    lm = jnp.where(col < NLAB, logits, jnp.float32(-1e30))
    mx = jnp.max(lm, axis=1, keepdims=True)
    lse = jnp.log(jnp.sum(jnp.exp(lm - mx), axis=1, keepdims=True)) + mx
    tgt = tgt_ref[0, 0, :]
    tlogit = jnp.sum(
        jnp.where(col == tgt[:, None], lm, jnp.float32(0.0)),
        axis=1, keepdims=True,
    )
    loss_part = jnp.sum(lse - tlogit)
    pred = jnp.min(jnp.where(lm >= mx, col, NLABP), axis=1)
    cnt_part = jnp.sum((pred == tgt).astype(jnp.float32))

    @pl.when(i == 0)
    def _():
        loss_ref[...] = jnp.zeros_like(loss_ref)
        cnt_ref[...] = jnp.zeros_like(cnt_ref)

    loss_ref[...] += loss_part
    cnt_ref[...] += cnt_part


def _make_head(interpret=False):
    return pl.pallas_call(
        _head_body,
        grid=(NBLK,),
        in_specs=[
            pl.BlockSpec((BLK, DP), lambda i: (i, 0)),
            pl.BlockSpec((DP, DH), lambda i: (0, 0)),
            pl.BlockSpec((1, DH), lambda i: (0, 0)),
            pl.BlockSpec((DH, NLABP), lambda i: (0, 0)),
            pl.BlockSpec((1, NLABP), lambda i: (0, 0)),
            pl.BlockSpec((1, 1, BLK), lambda i: (i, 0, 0)),
        ],
        out_specs=[
            pl.BlockSpec((1, 1), lambda i: (0, 0)),
            pl.BlockSpec((1, 1), lambda i: (0, 0)),
        ],
        out_shape=[
            jax.ShapeDtypeStruct((1, 1), jnp.float32),
            jax.ShapeDtypeStruct((1, 1), jnp.float32),
        ],
        interpret=interpret,
    )


@jax.jit
def kernel(surrounds, centers, targets, emb_surround, emb_center,
           W_h, b_h, W_o, b_o):
    sidx = surrounds.astype(jnp.int32).reshape(NW, NCHUNK, SPC * L)
    cidx = centers.astype(jnp.int32).reshape(NW, NCHUNK, SPC)

    sur_p, cen_p = _make_pad()(emb_surround, emb_center)
    pooled = _make_sc_pool()(sur_p, cen_p, sidx, cidx)

    wh = jnp.pad(W_h, ((0, DP - D), (0, 0)))
    wo = jnp.pad(W_o, ((0, 0), (0, NLABP - NLAB)))
    bo = jnp.pad(b_o, (0, NLABP - NLAB))
    tgt = targets.astype(jnp.int32).reshape(NBLK, 1, BLK)

    loss, cnt = _make_head()(
        pooled, wh, b_h[None, :], wo, bo[None, :], tgt
    )
    return loss[0, 0], cnt[0, 0] / jnp.float32(B)
